# baseline (speedup 1.0000x reference)
import functools

import jax
import jax.numpy as jnp
from jax import lax
from jax.experimental import pallas as pl
from jax.experimental.pallas import tpu as pltpu
from jax.experimental.pallas import tpu_sc as plsc

VOCAB = 100000
FRONT = 256
EMBED = 64

_NC = 2
_NS = 16
_NW = _NC * _NS


def _table_transform(table, W1, b1, W2, b2):
    BM = 2000
    grid = (VOCAB // BM,)

    def body(x_ref, w1_ref, b1_ref, w2_ref, b2_ref, o_ref):
        x = x_ref[:]
        h = lax.dot_general(x, w1_ref[:], (((1,), (1,)), ((), ())),
                            preferred_element_type=jnp.float32) + b1_ref[:]
        g = h * 0.5 * (1.0 + lax.erf(h * (2.0 ** -0.5)))
        o_ref[:] = lax.dot_general(g, w2_ref[:], (((1,), (1,)), ((), ())),
                                   preferred_element_type=jnp.float32) + b2_ref[:]

    return pl.pallas_call(
        body,
        grid=grid,
        in_specs=[
            pl.BlockSpec((BM, FRONT), lambda i: (i, 0)),
            pl.BlockSpec((EMBED, FRONT), lambda i: (0, 0)),
            pl.BlockSpec((1, EMBED), lambda i: (0, 0)),
            pl.BlockSpec((EMBED, EMBED), lambda i: (0, 0)),
            pl.BlockSpec((1, EMBED), lambda i: (0, 0)),
        ],
        out_specs=pl.BlockSpec((BM, EMBED), lambda i: (i, 0)),
        out_shape=jax.ShapeDtypeStruct((VOCAB, EMBED), jnp.float32),
    )(table, W1, b1.reshape(1, EMBED), W2, b2.reshape(1, EMBED))


def _sc_gather(t2, idx_flat, B, L):
    total = idx_flat.shape[0]
    b_per_w = total // _NW
    S = 32
    C = S * L
    n_chunks = b_per_w // C
    s_per_w = B // _NW

    mesh = plsc.VectorSubcoreMesh(core_axis_name="c", subcore_axis_name="s")

    @functools.partial(
        pl.kernel,
        mesh=mesh,
        out_type=jax.ShapeDtypeStruct((B, L, EMBED), jnp.float32),
        scratch_types=[
            pltpu.VMEM((C,), jnp.int32),
            pltpu.VMEM((C, EMBED), jnp.float32),
            pltpu.SemaphoreType.DMA,
            pltpu.SemaphoreType.DMA,
        ],
        compiler_params=pltpu.CompilerParams(use_tc_tiling_on_sc=False),
    )
    def k(t2_hbm, idx_hbm, out_hbm, idx_v, rows_v, gsem, osem):
        wid = lax.axis_index("s") * _NC + lax.axis_index("c")
        base = wid * b_per_w
        sbase = wid * s_per_w
        for c in range(n_chunks):
            pltpu.sync_copy(idx_hbm.at[pl.ds(base + c * C, C)], idx_v)
            pltpu.async_copy(t2_hbm.at[idx_v], rows_v, gsem).wait()
            cps = [
                pltpu.async_copy(rows_v.at[pl.ds(s * L, L)],
                                 out_hbm.at[sbase + c * S + s], osem)
                for s in range(S)
            ]
            for cp in cps:
                cp.wait()

    return k(t2, idx_flat)


def kernel(idxs, table, W1, b1, W2, b2):
    B, L = idxs.shape
    t2 = _table_transform(table, W1, b1, W2, b2)
    return _sc_gather(t2, idxs.reshape(-1).astype(jnp.int32), B, L)


# speedup vs baseline: 1.2349x; 1.2349x over previous
import functools

import jax
import jax.numpy as jnp
from jax import lax
from jax.experimental import pallas as pl
from jax.experimental.pallas import tpu as pltpu
from jax.experimental.pallas import tpu_sc as plsc

VOCAB = 100000
FRONT = 256
EMBED = 64

_NC = 2
_NS = 16
_NW = _NC * _NS


def _table_transform(table, W1, b1, W2, b2):
    BM = 2000
    HALF = VOCAB // 2
    grid = (HALF // BM,)

    def body(xlo_ref, xhi_ref, w1_ref, b1_ref, w2_ref, b2_ref, o_ref):
        def f(x):
            h = lax.dot_general(x, w1_ref[:], (((1,), (1,)), ((), ())),
                                preferred_element_type=jnp.float32) + b1_ref[:]
            g = h * 0.5 * (1.0 + lax.erf(h * (2.0 ** -0.5)))
            return lax.dot_general(g, w2_ref[:], (((1,), (1,)), ((), ())),
                                   preferred_element_type=jnp.float32) + b2_ref[:]
        o_ref[:, 0:EMBED] = f(xlo_ref[:])
        o_ref[:, EMBED:2 * EMBED] = f(xhi_ref[:])

    return pl.pallas_call(
        body,
        grid=grid,
        in_specs=[
            pl.BlockSpec((BM, FRONT), lambda i: (i, 0)),
            pl.BlockSpec((BM, FRONT), lambda i: (i + HALF // BM, 0)),
            pl.BlockSpec((EMBED, FRONT), lambda i: (0, 0)),
            pl.BlockSpec((1, EMBED), lambda i: (0, 0)),
            pl.BlockSpec((EMBED, EMBED), lambda i: (0, 0)),
            pl.BlockSpec((1, EMBED), lambda i: (0, 0)),
        ],
        out_specs=pl.BlockSpec((BM, 2 * EMBED), lambda i: (i, 0)),
        out_shape=jax.ShapeDtypeStruct((HALF, 2 * EMBED), jnp.float32),
    )(table, table, W1, b1.reshape(1, EMBED), W2, b2.reshape(1, EMBED))


def _sc_gather(t2, idx_flat, B, L):
    total = idx_flat.shape[0]
    b_per_w = total // _NW
    S = 32
    C = S * L
    n_chunks = b_per_w // C
    s_per_w = B // _NW

    mesh = plsc.VectorSubcoreMesh(core_axis_name="c", subcore_axis_name="s")

    @functools.partial(
        pl.kernel,
        mesh=mesh,
        out_type=jax.ShapeDtypeStruct((B, L, EMBED), jnp.float32),
        scratch_types=[
            pltpu.VMEM((C,), jnp.int32),
            pltpu.VMEM((C, EMBED), jnp.float32),
            pltpu.SemaphoreType.DMA,
            pltpu.SemaphoreType.DMA,
        ],
        compiler_params=pltpu.CompilerParams(use_tc_tiling_on_sc=False),
    )
    def k(t2_hbm, idx_hbm, out_hbm, idx_v, rows_v, gsem, osem):
        wid = lax.axis_index("s") * _NC + lax.axis_index("c")
        base = wid * b_per_w
        sbase = wid * s_per_w
        for c in range(n_chunks):
            pltpu.sync_copy(idx_hbm.at[pl.ds(base + c * C, C)], idx_v)
            pltpu.async_copy(t2_hbm.at[idx_v], rows_v, gsem).wait()
            cps = [
                pltpu.async_copy(rows_v.at[pl.ds(s * L, L)],
                                 out_hbm.at[sbase + c * S + s], osem)
                for s in range(S)
            ]
            for cp in cps:
                cp.wait()

    return k(t2, idx_flat)


def kernel(idxs, table, W1, b1, W2, b2):
    B, L = idxs.shape
    t2w = _table_transform(table, W1, b1, W2, b2)
    t2 = t2w.reshape(VOCAB, EMBED)
    v = idxs.reshape(-1).astype(jnp.int32)
    half = VOCAB // 2
    idx_r = 2 * jnp.where(v < half, v, v - half) + (v >= half).astype(jnp.int32)
    return _sc_gather(t2, idx_r, B, L)
